# Initial kernel scaffold; baseline (speedup 1.0000x reference)
import jax
import jax.numpy as jnp
from jax.experimental import pallas as pl


def kernel(FeatureT, centerInit, num1, num2):
    raise NotImplementedError("write your pallas kernel here")


# speedup vs baseline: 3.3547x; 3.3547x over previous
import jax
import jax.numpy as jnp
from jax.experimental import pallas as pl
from jax.experimental.pallas import tpu as pltpu

_B, _C, _N = 4, 768, 4096


def _tc_body(nums_ref, x_ref, c_ref,
             acc_ref, lab_ref, oh_ref, w_ref, d_ref, labp_ref, cini_ref):
    b = pl.program_id(0)
    x = x_ref[0]
    c = c_ref[...]
    cnorm = jnp.sqrt(jnp.sum(c * c, axis=1, keepdims=True))
    cn = c / jnp.maximum(cnorm, 1e-12)
    ss = jnp.sum(x * x, axis=0, keepdims=True)
    an = x / jnp.maximum(jnp.sqrt(ss), 1e-12)
    g = jax.lax.dot_general(cn, an, (((1,), (0,)), ((), ())),
                            preferred_element_type=jnp.float32)
    d = 0.5 * (1.0 - g)
    d0, d1 = d[0:1, :], d[1:2, :]
    labf = (d1 < d0).astype(jnp.float32)
    th1 = d1 * labf
    th0 = d0 * (1.0 - labf)
    cnt1 = jnp.sum(labf)
    chg_mean = jnp.sum(th1) / (cnt1 + 1.0)
    unchg_mean = jnp.sum(th0) / (_N - cnt1 + 1.0)
    num1 = nums_ref[0]
    num2 = nums_ref[1]
    drop = jnp.logical_or(th1 > chg_mean / num1, th0 > unchg_mean * num2)
    keep = 1.0 - drop.astype(jnp.float32)
    m = jnp.concatenate([(1.0 - labf) * keep, labf * keep], axis=0)
    fbfilt = jax.lax.dot_general(m, x, (((1,), (1,)), ((), ())),
                                 preferred_element_type=jnp.float32)
    cnum = jnp.sum(m, axis=1, keepdims=True) + 1.0
    citer = fbfilt / cnum

    @pl.when(b == 0)
    def _():
        acc_ref[...] = jnp.zeros_like(acc_ref)

    acc_ref[...] += citer * (1.0 / _B)

    lab_i = labf.astype(jnp.int32)
    lab_ref[0] = lab_i
    labp_ref[0] = lab_i
    oh_ref[0] = jnp.concatenate([1.0 - labf, labf], axis=0)
    d_ref[0] = d
    mn = jnp.min(d, axis=1, keepdims=True)
    mx = jnp.max(d, axis=1, keepdims=True)
    w_ref[0] = 1.0 - (d - mn) / (mx - mn + 1e-07)

    @pl.when(b == _B - 1)
    def _():
        num = jnp.sum(citer * c, axis=1, keepdims=True)
        na = jnp.sqrt(jnp.sum(citer * citer, axis=1, keepdims=True))
        nb = jnp.sqrt(jnp.sum(c * c, axis=1, keepdims=True))
        val = num / (jnp.maximum(na, 1e-8) * jnp.maximum(nb, 1e-8))
        cini_ref[...] = jnp.sum(val, axis=0, keepdims=True) / _B


def kernel(FeatureT, centerInit, num1, num2):
    ft = FeatureT.reshape(_B, _C, _N)
    nums = jnp.stack([jnp.asarray(num1), jnp.asarray(num2)]).astype(jnp.float32)
    f32, i32 = jnp.float32, jnp.int32
    acc, lab3, oh, w, dd, labp3, cini = pl.pallas_call(
        _tc_body,
        grid=(_B,),
        in_specs=[
            pl.BlockSpec(memory_space=pltpu.SMEM),
            pl.BlockSpec((1, _C, _N), lambda b: (b, 0, 0)),
            pl.BlockSpec((2, _C), lambda b: (0, 0)),
        ],
        out_specs=[
            pl.BlockSpec((2, _C), lambda b: (0, 0)),
            pl.BlockSpec((1, 1, _N), lambda b: (b, 0, 0)),
            pl.BlockSpec((1, 2, _N), lambda b: (b, 0, 0)),
            pl.BlockSpec((1, 2, _N), lambda b: (b, 0, 0)),
            pl.BlockSpec((1, 2, _N), lambda b: (b, 0, 0)),
            pl.BlockSpec((1, 1, _N), lambda b: (b, 0, 0)),
            pl.BlockSpec((1, 1), lambda b: (0, 0)),
        ],
        out_shape=[
            jax.ShapeDtypeStruct((2, _C), f32),
            jax.ShapeDtypeStruct((_B, 1, _N), i32),
            jax.ShapeDtypeStruct((_B, 2, _N), f32),
            jax.ShapeDtypeStruct((_B, 2, _N), f32),
            jax.ShapeDtypeStruct((_B, 2, _N), f32),
            jax.ShapeDtypeStruct((_B, 1, _N), i32),
            jax.ShapeDtypeStruct((1, 1), f32),
        ],
    )(nums, ft, centerInit)
    labels = lab3.reshape(_B, _N)
    labelP = labp3.reshape(_B, _N)
    onehot = jnp.transpose(oh, (0, 2, 1))
    Weight = jnp.transpose(w, (0, 2, 1))
    dist = jnp.transpose(dd, (0, 2, 1))
    return acc, [labels, onehot, Weight, dist, labelP], cini[0, 0]


# speedup vs baseline: 3.3593x; 1.0014x over previous
import jax
import jax.numpy as jnp
from jax.experimental import pallas as pl
from jax.experimental.pallas import tpu as pltpu

_B, _C, _N = 4, 768, 4096


def _tc_body(nums_ref, x_ref, c_ref,
             acc_ref, lab_ref, oh_ref, w_ref, d_ref, cini_ref):
    b = pl.program_id(0)
    x = x_ref[0]
    c = c_ref[...]
    cnorm = jnp.sqrt(jnp.sum(c * c, axis=1, keepdims=True))
    cn = c / jnp.maximum(cnorm, 1e-12)
    ss = jnp.sum(x * x, axis=0, keepdims=True)
    an = x / jnp.maximum(jnp.sqrt(ss), 1e-12)
    g = jax.lax.dot_general(cn, an, (((1,), (0,)), ((), ())),
                            preferred_element_type=jnp.float32)
    d = 0.5 * (1.0 - g)
    d0, d1 = d[0:1, :], d[1:2, :]
    labf = (d1 < d0).astype(jnp.float32)
    th1 = d1 * labf
    th0 = d0 * (1.0 - labf)
    cnt1 = jnp.sum(labf)
    chg_mean = jnp.sum(th1) / (cnt1 + 1.0)
    unchg_mean = jnp.sum(th0) / (_N - cnt1 + 1.0)
    num1 = nums_ref[0]
    num2 = nums_ref[1]
    drop = jnp.logical_or(th1 > chg_mean / num1, th0 > unchg_mean * num2)
    keep = 1.0 - drop.astype(jnp.float32)
    m = jnp.concatenate([(1.0 - labf) * keep, labf * keep], axis=0)
    fbfilt = jax.lax.dot_general(m, x, (((1,), (1,)), ((), ())),
                                 preferred_element_type=jnp.float32)
    cnum = jnp.sum(m, axis=1, keepdims=True) + 1.0
    citer = fbfilt / cnum

    @pl.when(b == 0)
    def _():
        acc_ref[...] = jnp.zeros_like(acc_ref)

    acc_ref[...] += citer * (1.0 / _B)

    lab_ref[0] = labf.astype(jnp.int32)
    oh_ref[0] = jnp.concatenate([1.0 - labf, labf], axis=0)
    d_ref[0] = d
    mn = jnp.min(d, axis=1, keepdims=True)
    mx = jnp.max(d, axis=1, keepdims=True)
    w_ref[0] = 1.0 - (d - mn) / (mx - mn + 1e-07)

    @pl.when(b == _B - 1)
    def _():
        num = jnp.sum(citer * c, axis=1, keepdims=True)
        na = jnp.sqrt(jnp.sum(citer * citer, axis=1, keepdims=True))
        nb = jnp.sqrt(jnp.sum(c * c, axis=1, keepdims=True))
        val = num / (jnp.maximum(na, 1e-8) * jnp.maximum(nb, 1e-8))
        cini_ref[...] = jnp.sum(val, axis=0, keepdims=True) / _B


def kernel(FeatureT, centerInit, num1, num2):
    ft = FeatureT.reshape(_B, _C, _N)
    nums = jnp.stack([jnp.asarray(num1), jnp.asarray(num2)]).astype(jnp.float32)
    f32, i32 = jnp.float32, jnp.int32
    acc, lab3, oh, w, dd, cini = pl.pallas_call(
        _tc_body,
        grid=(_B,),
        in_specs=[
            pl.BlockSpec(memory_space=pltpu.SMEM),
            pl.BlockSpec((1, _C, _N), lambda b: (b, 0, 0)),
            pl.BlockSpec((2, _C), lambda b: (0, 0)),
        ],
        out_specs=[
            pl.BlockSpec((2, _C), lambda b: (0, 0)),
            pl.BlockSpec((1, 1, _N), lambda b: (b, 0, 0)),
            pl.BlockSpec((1, 2, _N), lambda b: (b, 0, 0)),
            pl.BlockSpec((1, 2, _N), lambda b: (b, 0, 0)),
            pl.BlockSpec((1, 2, _N), lambda b: (b, 0, 0)),
            pl.BlockSpec((1, 1), lambda b: (0, 0)),
        ],
        out_shape=[
            jax.ShapeDtypeStruct((2, _C), f32),
            jax.ShapeDtypeStruct((_B, 1, _N), i32),
            jax.ShapeDtypeStruct((_B, 2, _N), f32),
            jax.ShapeDtypeStruct((_B, 2, _N), f32),
            jax.ShapeDtypeStruct((_B, 2, _N), f32),
            jax.ShapeDtypeStruct((1, 1), f32),
        ],
    )(nums, ft, centerInit)
    labels = lab3.reshape(_B, _N)
    onehot = jnp.transpose(oh, (0, 2, 1))
    Weight = jnp.transpose(w, (0, 2, 1))
    dist = jnp.transpose(dd, (0, 2, 1))
    return acc, [labels, onehot, Weight, dist, labels], cini[0, 0]
